# Initial kernel scaffold; baseline (speedup 1.0000x reference)
import jax
import jax.numpy as jnp
from jax.experimental import pallas as pl


def kernel(x, edge_index, edge_attr, batch, params):
    raise NotImplementedError("write your pallas kernel here")


# speedup vs baseline: 4.9004x; 4.9004x over previous
import functools
import math

import jax
import jax.numpy as jnp
from jax import lax
from jax.experimental import pallas as pl
from jax.experimental.pallas import tpu as pltpu
from jax.experimental.pallas import tpu_sc as plsc

_NC = 2
_NS = 16
_NW = _NC * _NS
_K = 128
_NBUF = 4


def _sc_segsum(table, srcp, dstp, n_acc, eap=None):
    chunks = srcp.shape[1]
    C = table.shape[1]
    D = _NBUF
    NI = 2 * _NBUF
    rpt = n_acc // _NS
    assert n_acc % (_NS * 8) == 0 and chunks % 8 == 0

    zeros_c = jnp.zeros((rpt, C), jnp.float32)
    with_ea = eap is not None

    out_type = [jax.ShapeDtypeStruct((_NC, n_acc, C), jnp.float32)]
    scratch = [pltpu.VMEM((NI, _K), jnp.int32),
               pltpu.VMEM((D, _K), jnp.int32)]
    scratch += [pltpu.VMEM((_K, C), jnp.float32) for _ in range(D)]
    scratch += [pltpu.SemaphoreType.DMA for _ in range(NI)]
    scratch += [pltpu.SemaphoreType.DMA for _ in range(D)]
    scratch += [pltpu.SemaphoreType.DMA for _ in range(D)]
    scratch += [pltpu.VMEM_SHARED((n_acc, C), jnp.float32)]
    inputs = [table, srcp, dstp, zeros_c]
    if with_ea:
        zeros_e = jnp.zeros((rpt, 8), jnp.float32)
        inputs += [eap, zeros_e]
        out_type += [jax.ShapeDtypeStruct((_NC, n_acc, 8), jnp.float32)]
        scratch += [pltpu.VMEM_SHARED((n_acc, 8), jnp.float32)]
        scratch += [pltpu.VMEM((_K, 8), jnp.float32) for _ in range(D)]
        scratch += [pltpu.SemaphoreType.DMA for _ in range(D)]

    mesh = plsc.VectorSubcoreMesh(core_axis_name="c", subcore_axis_name="s")

    def body(*refs):
        it = iter(refs)
        table_h = next(it)
        src_h = next(it)
        dst_h = next(it)
        zc_h = next(it)
        if with_ea:
            ea_h = next(it)
            ze_h = next(it)
        out_h = next(it)
        if with_ea:
            oute_h = next(it)
        sbuf = next(it)
        dbuf = next(it)
        rbufs = [next(it) for _ in range(D)]
        isems = [next(it) for _ in range(NI)]
        dsems = [next(it) for _ in range(D)]
        gsems = [next(it) for _ in range(D)]
        acc = next(it)
        if with_ea:
            acc_e = next(it)
            ebufs = [next(it) for _ in range(D)]
            esems = [next(it) for _ in range(D)]

        c = lax.axis_index("c")
        s = lax.axis_index("s")
        wid = c * _NS + s
        r0 = s * rpt
        my_src = src_h.at[wid]
        my_dst = dst_h.at[wid]

        pltpu.sync_copy(zc_h, acc.at[pl.ds(r0, rpt)])
        if with_ea:
            pltpu.sync_copy(ze_h, acc_e.at[pl.ds(r0, rpt)])
        plsc.subcore_barrier()

        for b in range(NI):
            pltpu.async_copy(my_src.at[b], sbuf.at[b], isems[b])

        def iteration(j, b4, b8):
            jc = j - D
            b8c = (b8 + NI - D) % NI

            @pl.when((jc >= 0) & (jc < chunks))
            def _():
                pltpu.make_async_copy(table_h.at[sbuf.at[b8c]],
                                      rbufs[b4], gsems[b4]).wait()
                pltpu.make_async_copy(my_dst.at[jc], dbuf.at[b4],
                                      dsems[b4]).wait()
                pltpu.sync_copy(rbufs[b4], acc.at[dbuf.at[b4]], add=True)
                if with_ea:
                    pltpu.make_async_copy(ea_h.at[wid].at[jc], ebufs[b4],
                                          esems[b4]).wait()
                    pltpu.sync_copy(ebufs[b4], acc_e.at[dbuf.at[b4]],
                                    add=True)
                @pl.when(jc + NI < chunks)
                def _():
                    pltpu.async_copy(my_src.at[jc + NI], sbuf.at[b8c],
                                     isems[b8c])

            @pl.when(j < chunks)
            def _():
                pltpu.async_copy(my_dst.at[j], dbuf.at[b4], dsems[b4])
                if with_ea:
                    pltpu.async_copy(ea_h.at[wid].at[j], ebufs[b4],
                                     esems[b4])
                pltpu.make_async_copy(my_src.at[j], sbuf.at[b8],
                                      isems[b8]).wait()
                pltpu.async_copy(table_h.at[sbuf.at[b8]], rbufs[b4],
                                 gsems[b4])

        def outer(g, carry):
            for u in range(NI):
                j = g * NI + u
                iteration(j, u % D, u % NI)
            return carry

        lax.fori_loop(0, (chunks + NI) // NI, outer, 0)
        plsc.subcore_barrier()
        pltpu.sync_copy(acc.at[pl.ds(r0, rpt)],
                        out_h.at[c].at[pl.ds(r0, rpt)])
        if with_ea:
            pltpu.sync_copy(acc_e.at[pl.ds(r0, rpt)],
                            oute_h.at[c].at[pl.ds(r0, rpt)])

    f = pl.kernel(body, out_type=tuple(out_type), mesh=mesh,
                  scratch_types=scratch,
                  compiler_params=pltpu.CompilerParams(
                      use_tc_tiling_on_sc=False))
    res = f(*inputs)
    return res if with_ea else res[0]


_R = 2000


def _layer_tc(A0, A1, E0, E1, h, Wh, We8, Wr, br, gamma, beta):
    N, Ci = h.shape
    Co = Wh.shape[1]
    nb = N // _R
    inv_n = 1.0 / N

    def ka(a0, a1, e0, e1, hr, wh, we, wr, brr, u_out, st_out, acc):
        i = pl.program_id(0)

        @pl.when(i == 0)
        def _():
            acc[...] = jnp.zeros_like(acc)

        at = a0[...] + a1[...]
        ea = e0[...] + e1[...]
        u = (jnp.dot(at, wh[...], preferred_element_type=jnp.float32, precision=lax.Precision.HIGHEST)
             + jnp.dot(ea, we[...], preferred_element_type=jnp.float32, precision=lax.Precision.HIGHEST)
             + jnp.dot(hr[...], wr[...], preferred_element_type=jnp.float32, precision=lax.Precision.HIGHEST)
             + brr[...])
        u_out[...] = u
        acc[0:1, :] += jnp.sum(u, axis=0, keepdims=True)
        acc[1:2, :] += jnp.sum(u * u, axis=0, keepdims=True)

        @pl.when(i == nb - 1)
        def _():
            st_out[...] = acc[...]

    row = lambda w: pl.BlockSpec((_R, w), lambda i: (i, 0))
    full = lambda a: pl.BlockSpec(a.shape, lambda i: (0,) * a.ndim)
    u, st = pl.pallas_call(
        ka,
        grid=(nb,),
        in_specs=[row(Ci), row(Ci), row(8), row(8), row(Ci),
                  full(Wh), full(We8), full(Wr), full(br)],
        out_specs=[row(Co), pl.BlockSpec((8, Co), lambda i: (0, 0))],
        out_shape=[jax.ShapeDtypeStruct((N, Co), jnp.float32),
                   jax.ShapeDtypeStruct((8, Co), jnp.float32)],
        scratch_shapes=[pltpu.VMEM((8, Co), jnp.float32)],
    )(A0, A1, E0, E1, h, Wh, We8, Wr, br)

    def kb(ur, str_, g_, b_, h_out):
        mu = str_[0:1, :] * inv_n
        var = str_[1:2, :] * inv_n - mu * mu
        inv = lax.rsqrt(var + 1e-5)
        h_out[...] = jax.nn.gelu((ur[...] - mu) * inv * g_[...] + b_[...])

    h_out = pl.pallas_call(
        kb,
        grid=(nb,),
        in_specs=[row(Co), pl.BlockSpec((8, Co), lambda i: (0, 0)),
                  full(gamma), full(beta)],
        out_specs=row(Co),
        out_shape=jax.ShapeDtypeStruct((N, Co), jnp.float32),
    )(u, st, gamma, beta)
    return h_out


def _heads_tc(h4, batch3, zw, pw, emw, elw, ng):
    N = h4.shape[0]
    nb = N // _R
    zd = zw["W2a"].shape[1]
    eta = emw["W2"].shape[1]

    def body(hr, br_, W1, b1, W2a, b2a, W2b, b2b, Wsa, Wsb,
             Wp, bp, mW1, mb1, mW2, mb2, mWs, lW1, lb1, lW2, lb2, lWs,
             zmu_o, zsg_o, emu_o, esg_o, pool_acc, cnt_acc):
        i = pl.program_id(0)

        @pl.when(i == 0)
        def _():
            pool_acc[...] = jnp.zeros_like(pool_acc)
            cnt_acc[...] = jnp.zeros_like(cnt_acc)

        hb = hr[...]
        hh = jax.nn.gelu(jnp.dot(hb, W1[...],
                                 preferred_element_type=jnp.float32, precision=lax.Precision.HIGHEST) + b1[...])
        zmu_o[...] = (jnp.dot(hh, W2a[...], preferred_element_type=jnp.float32, precision=lax.Precision.HIGHEST)
                      + b2a[...]
                      + jnp.dot(hb, Wsa[...],
                                preferred_element_type=jnp.float32, precision=lax.Precision.HIGHEST))
        zs = (jnp.dot(hh, W2b[...], preferred_element_type=jnp.float32, precision=lax.Precision.HIGHEST)
              + b2b[...]
              + jnp.dot(hb, Wsb[...], preferred_element_type=jnp.float32, precision=lax.Precision.HIGHEST))
        zsg_o[...] = jnp.exp(jnp.clip(zs, -30.0, 20.0))

        bt = br_[0]
        oh = (bt == lax.broadcasted_iota(jnp.int32, (ng, 1), 0)
              ).astype(jnp.float32)
        pool_acc[...] += jnp.dot(oh, hb, preferred_element_type=jnp.float32, precision=lax.Precision.HIGHEST)
        cnt_acc[...] += jnp.dot(oh, jnp.ones((_R, 128), jnp.float32),
                                preferred_element_type=jnp.float32, precision=lax.Precision.HIGHEST)

        @pl.when(i == nb - 1)
        def _():
            pooled = pool_acc[...] / jnp.maximum(cnt_acc[...], 1.0)
            g = jnp.dot(pooled, Wp[...],
                        preferred_element_type=jnp.float32, precision=lax.Precision.HIGHEST) + bp[...]

            def rb(w1, bb1, w2, bb2, ws):
                t = jax.nn.gelu(jnp.dot(g, w1[...],
                                        preferred_element_type=jnp.float32, precision=lax.Precision.HIGHEST)
                                + bb1[...])
                return (jnp.dot(t, w2[...], preferred_element_type=jnp.float32, precision=lax.Precision.HIGHEST)
                        + bb2[...]
                        + jnp.dot(g, ws[...],
                                  preferred_element_type=jnp.float32, precision=lax.Precision.HIGHEST))

            emu_o[...] = rb(mW1, mb1, mW2, mb2, mWs)
            esg_o[...] = jnp.exp(jnp.clip(rb(lW1, lb1, lW2, lb2, lWs),
                                          -30.0, 20.0))

    row = lambda w: pl.BlockSpec((_R, w), lambda i: (i, 0))
    full = lambda a: pl.BlockSpec(a.shape, lambda i: (0,) * a.ndim)
    const = lambda shp: pl.BlockSpec(shp, lambda i: (0,) * len(shp))
    weights = [zw["W1"], zw["b1"], zw["W2a"], zw["b2a"], zw["W2b"], zw["b2b"],
               zw["Wsa"], zw["Wsb"], pw["W"], pw["b"],
               emw["W1"], emw["b1"], emw["W2"], emw["b2"], emw["Ws"],
               elw["W1"], elw["b1"], elw["W2"], elw["b2"], elw["Ws"]]
    return pl.pallas_call(
        body,
        grid=(nb,),
        in_specs=[row(128), pl.BlockSpec((1, 1, _R), lambda i: (i, 0, 0))]
                 + [full(w) for w in weights],
        out_specs=[row(zd), row(zd), const((ng, eta)), const((ng, eta))],
        out_shape=[jax.ShapeDtypeStruct((N, zd), jnp.float32),
                   jax.ShapeDtypeStruct((N, zd), jnp.float32),
                   jax.ShapeDtypeStruct((ng, eta), jnp.float32),
                   jax.ShapeDtypeStruct((ng, eta), jnp.float32)],
        scratch_shapes=[pltpu.VMEM((ng, 128), jnp.float32),
                        pltpu.VMEM((ng, 128), jnp.float32)],
    )(h4, batch3, *weights)


def kernel(x, edge_index, edge_attr, batch, params):
    N, _ = x.shape
    E = edge_index.shape[1]
    ng = 256
    src = edge_index[0].astype(jnp.int32)
    dst = edge_index[1].astype(jnp.int32)

    chunks = 8 * (-(-E // (_NW * _K * 8)))
    epad = _NW * _K * chunks - E
    srcp = jnp.concatenate([src, jnp.zeros((epad,), jnp.int32)]
                           ).reshape(_NW, chunks, _K)
    dstp = jnp.concatenate([dst, jnp.full((epad,), N, jnp.int32)]
                           ).reshape(_NW, chunks, _K)
    ea8 = jnp.concatenate(
        [edge_attr.astype(jnp.float32), jnp.ones((E, 1), jnp.float32),
         jnp.zeros((E, 3), jnp.float32)], axis=1)
    eap = jnp.concatenate([ea8, jnp.zeros((epad, 8), jnp.float32)]
                          ).reshape(_NW, chunks, _K, 8)
    n_acc = (_NS * 8) * (-(-(N + 1) // (_NS * 8)))

    def we8(p, ci):
        return jnp.concatenate(
            [p["W_msg"][ci:ci + 4], p["b_msg"][None, :],
             jnp.zeros((3, p["W_msg"].shape[1]), jnp.float32)], axis=0)

    ax, ae = _sc_segsum(x, srcp, dstp, n_acc, eap=eap)
    E0, E1 = ae[0, :N], ae[1, :N]

    h = x
    for i in range(4):
        p = params["conv%d" % i]
        ci = h.shape[1]
        if i == 0:
            a = ax
        elif ci <= 32:
            a = _sc_segsum(h, srcp, dstp, n_acc)
        else:
            aa = _sc_segsum(h[:, :32], srcp, dstp, n_acc)
            ab = _sc_segsum(h[:, 32:], srcp, dstp, n_acc)
            a = jnp.concatenate([aa, ab], axis=2)
        h = _layer_tc(a[0, :N], a[1, :N], E0, E1, h,
                      p["W_msg"][:ci], we8(p, ci), p["W_root"],
                      p["b_root"][None, :], p["gamma"][None, :],
                      p["beta"][None, :])

    zp = params["z_head"]
    zd = zp["W2"].shape[1] // 2
    zw = {"W1": zp["W1"], "b1": zp["b1"][None, :],
          "W2a": zp["W2"][:, :zd], "b2a": zp["b2"][None, :zd],
          "W2b": zp["W2"][:, zd:], "b2b": zp["b2"][None, zd:],
          "Wsa": zp["Wskip"][:, :zd], "Wsb": zp["Wskip"][:, zd:]}
    pw = {"W": params["pool"]["W"], "b": params["pool"]["b"][None, :]}

    def rbw(p):
        return {"W1": p["W1"], "b1": p["b1"][None, :], "W2": p["W2"],
                "b2": p["b2"][None, :], "Ws": p["Wskip"]}

    batch3 = batch.astype(jnp.int32).reshape(N // _R, 1, _R)
    z_mu, z_sigma, eta_mu, eta_sigma = _heads_tc(
        h, batch3, zw, pw, rbw(params["eta_mu"]), rbw(params["eta_ls"]), ng)
    return (z_mu, z_sigma, eta_mu, eta_sigma)
